# baseline (speedup 1.0000x reference)
import functools

import jax
import jax.numpy as jnp
from jax import lax
from jax.experimental import pallas as pl
from jax.experimental.pallas import tpu as pltpu
from jax.experimental.pallas import tpu_sc as plsc

_EMBED = 64
_LANES = 16


def _gather_add(flat_idx, token_table, pe2, *, n_workers, chunk, n_chunks):
    total = flat_idx.shape[0]
    per_w = total // n_workers
    mesh = plsc.VectorSubcoreMesh(core_axis_name="c", subcore_axis_name="s")

    @functools.partial(
        pl.kernel,
        mesh=mesh,
        compiler_params=pltpu.CompilerParams(use_tc_tiling_on_sc=False),
        out_type=jax.ShapeDtypeStruct((total, _EMBED), jnp.float32),
        scratch_types=[
            pltpu.VMEM((chunk,), jnp.int32),
            pltpu.VMEM((chunk, _EMBED), jnp.float32),
            pltpu.VMEM((chunk, _EMBED), jnp.float32),
            pltpu.SemaphoreType.DMA,
        ],
    )
    def k(idx_hbm, table_hbm, pe2_hbm, out_hbm, idx_v, rows_v, pe2_v, sem):
        wid = lax.axis_index("s") * 2 + lax.axis_index("c")
        base = wid * per_w
        pltpu.sync_copy(pe2_hbm, pe2_v)

        def chunk_body(g, _):
            off = base + g * chunk
            pltpu.sync_copy(idx_hbm.at[pl.ds(off, chunk)], idx_v)
            pltpu.async_copy(table_hbm.at[idx_v], rows_v, sem).wait()

            def add_body(i, _):
                for d in range(_EMBED // _LANES):
                    sl = pl.ds(d * _LANES, _LANES)
                    rows_v[i, sl] = rows_v[i, sl] + pe2_v[i, sl]
                return 0

            lax.fori_loop(0, chunk, add_body, 0)
            pltpu.sync_copy(rows_v, out_hbm.at[pl.ds(off, chunk)])
            return 0

        lax.fori_loop(0, n_chunks, chunk_body, 0)

    return k(flat_idx, token_table, pe2)


def kernel(sequence, token_table, pe):
    b, l = sequence.shape
    flat_idx = sequence.reshape(-1).astype(jnp.int32)
    pe_l = pe[:l]
    pe2 = jnp.concatenate([pe_l, pe_l], axis=0)
    n_workers = 32
    chunk = 2 * l
    per_w = (b * l) // n_workers
    n_chunks = per_w // chunk
    out = _gather_add(
        flat_idx, token_table, pe2,
        n_workers=n_workers, chunk=chunk, n_chunks=n_chunks,
    )
    return out.reshape(b, l, _EMBED)


# speedup vs baseline: 1.1172x; 1.1172x over previous
import functools

import jax
import jax.numpy as jnp
from jax import lax
from jax.experimental import pallas as pl
from jax.experimental.pallas import tpu as pltpu
from jax.experimental.pallas import tpu_sc as plsc

_EMBED = 64
_LANES = 16


def _gather_add(idx2d, token_table, pe2, *, n_workers, chunk, n_chunks):
    total = idx2d.shape[0] * idx2d.shape[1]
    chunks_per_w = n_chunks // n_workers
    mesh = plsc.VectorSubcoreMesh(core_axis_name="c", subcore_axis_name="s")

    @functools.partial(
        pl.kernel,
        mesh=mesh,
        compiler_params=pltpu.CompilerParams(use_tc_tiling_on_sc=False),
        out_type=jax.ShapeDtypeStruct((total, _EMBED), jnp.float32),
        scratch_types=[
            pltpu.VMEM((chunks_per_w, chunk), jnp.int32),
            pltpu.VMEM((chunk, _EMBED), jnp.float32),
            pltpu.VMEM((chunk, _EMBED), jnp.float32),
            pltpu.VMEM((chunk, _EMBED), jnp.float32),
            pltpu.SemaphoreType.DMA,
        ],
    )
    def k(idx_hbm, table_hbm, pe2_hbm, out_hbm, idx_v, rows_a, rows_b, pe2_v,
          sem):
        wid = lax.axis_index("s") * 2 + lax.axis_index("c")
        base = wid * chunks_per_w * chunk
        pltpu.sync_copy(pe2_hbm, pe2_v)
        pltpu.sync_copy(idx_hbm.at[pl.ds(wid * chunks_per_w, chunks_per_w)],
                        idx_v)

        def add_pe(buf):
            def add_body(i, _):
                for d in range(_EMBED // _LANES):
                    sl = pl.ds(d * _LANES, _LANES)
                    buf[i, sl] = buf[i, sl] + pe2_v[i, sl]
                return 0

            lax.fori_loop(0, chunk, add_body, 0)

        def start_gather(g, buf):
            return pltpu.async_copy(table_hbm.at[idx_v.at[g]], buf, sem)

        def wait_gather(buf):
            pltpu.make_async_copy(table_hbm.at[idx_v.at[0]], buf, sem).wait()

        def copy_out(g, buf):
            pltpu.sync_copy(buf, out_hbm.at[pl.ds(base + g * chunk, chunk)])

        start_gather(0, rows_a)

        def pair_body(j, _):
            a = 2 * j
            wait_gather(rows_a)
            start_gather(a + 1, rows_b)
            add_pe(rows_a)
            copy_out(a, rows_a)
            wait_gather(rows_b)

            @pl.when(j + 1 < chunks_per_w // 2)
            def _():
                start_gather(a + 2, rows_a)

            add_pe(rows_b)
            copy_out(a + 1, rows_b)
            return 0

        lax.fori_loop(0, chunks_per_w // 2, pair_body, 0)

    return k(idx2d, token_table, pe2)


def kernel(sequence, token_table, pe):
    b, l = sequence.shape
    chunk = 2 * l
    n_chunks = (b * l) // chunk
    idx2d = sequence.reshape(n_chunks, chunk).astype(jnp.int32)
    pe_l = pe[:l]
    pe2 = jnp.concatenate([pe_l, pe_l], axis=0)
    out = _gather_add(
        idx2d, token_table, pe2,
        n_workers=32, chunk=chunk, n_chunks=n_chunks,
    )
    return out.reshape(b, l, _EMBED)
